# baseline (speedup 1.0000x reference)
import jax
import jax.numpy as jnp
from jax import lax
from jax.experimental import pallas as pl
from jax.experimental.pallas import tpu as pltpu
from jax.experimental.pallas import tpu_sc as plsc

D = 64
NC = 2
NS = 16
NW = NC * NS
BW = 128
NBB = BW // 16


def _sc_body(seq_hbm, ids_hbm, table_hbm, out_hbm,
             idw_0, idw_1, rows1_0, rows1_1, rows2_0, rows2_1,
             seq_0, seq_1, outb_0, outb_1,
             sem_g0, sem_g1, sem_s0, sem_s1, sem_o0, sem_o1):
    idw_v = (idw_0, idw_1)
    rows1_v = (rows1_0, rows1_1)
    rows2_v = (rows2_0, rows2_1)
    seq_v = (seq_0, seq_1)
    out_v = (outb_0, outb_1)
    sem_g = (sem_g0, sem_g1)
    sem_s = (sem_s0, sem_s1)
    sem_o = (sem_o0, sem_o1)

    wid = lax.axis_index("s") * NC + lax.axis_index("c")
    wb = wid * BW
    nl = seq_hbm.shape[0]

    iota = lax.iota(jnp.int32, 16)

    def prefetch(l, b):
        @pl.when(l < nl)
        def _():
            pltpu.sync_copy(ids_hbm.at[l, :, :, pl.ds(wb, BW)], idw_v[b])
            pltpu.async_copy(table_hbm.at[idw_v[b].at[0, 0]], rows1_v[b],
                             sem_g[b])
            pltpu.async_copy(table_hbm.at[idw_v[b].at[1, 0]], rows2_v[b],
                             sem_g[b])
            pltpu.async_copy(seq_hbm.at[l, :, pl.ds(wb, BW)], seq_v[b],
                             sem_s[b])

    def wait_in(b):
        pltpu.make_async_copy(table_hbm.at[idw_v[b].at[0, 0]], rows1_v[b],
                              sem_g[b]).wait()
        pltpu.make_async_copy(table_hbm.at[idw_v[b].at[1, 0]], rows2_v[b],
                              sem_g[b]).wait()
        pltpu.make_async_copy(seq_hbm.at[0, :, pl.ds(0, BW)], seq_v[b],
                              sem_s[b]).wait()

    def wait_out(b):
        pltpu.make_async_copy(out_v[b], out_hbm.at[0, :, pl.ds(0, BW)],
                              sem_o[b]).wait()

    def compute(l, b):
        wait_in(b)

        @pl.when(l >= 2)
        def _():
            wait_out(b)

        r1, r2, sq, ob, iw = rows1_v[b], rows2_v[b], seq_v[b], out_v[b], idw_v[b]

        for bb in range(NBB):
            bsl = pl.ds(bb * 16, 16)
            w1f = iw[0, 1, bsl].astype(jnp.float32)
            w2f = iw[1, 1, bsl].astype(jnp.float32)
            pvec = iota + bb * 16

            @plsc.parallel_loop(0, D, unroll=4)
            def feat(d):
                dz = iota * 0 + d
                r1g = plsc.load_gather(r1, [pvec, dz])
                r2g = plsc.load_gather(r2, [pvec, dz])
                ob[d, bsl] = sq[d, bsl] + r1g * w1f + r2g * w2f

        pltpu.async_copy(ob, out_hbm.at[l, :, pl.ds(wb, BW)], sem_o[b])

    prefetch(0, 0)
    prefetch(1, 1)

    def body(tt, _):
        l = tt * 2
        compute(l, 0)
        prefetch(l + 2, 0)
        compute(l + 1, 1)
        prefetch(l + 3, 1)
        return 0

    lax.fori_loop(0, nl // 2, body, 0)

    wait_out(0)
    wait_out(1)


def kernel(sequence, id, player_embeddings):
    b, l, d = sequence.shape
    seq_t = jnp.transpose(sequence, (1, 2, 0))
    ids_t = jnp.transpose(id.astype(jnp.int32), (1, 2, 3, 0))
    table2 = jnp.pad(player_embeddings, ((0, 0), (0, 128 - d)))
    mesh = plsc.VectorSubcoreMesh(core_axis_name="c", subcore_axis_name="s")
    run = pl.kernel(
        _sc_body,
        out_type=jax.ShapeDtypeStruct((l, d, b), jnp.float32),
        mesh=mesh,
        compiler_params=pltpu.CompilerParams(needs_layout_passes=False),
        scratch_types=[
            pltpu.VMEM((2, 2, BW), jnp.int32),
            pltpu.VMEM((2, 2, BW), jnp.int32),
            pltpu.VMEM((BW, 128), jnp.float32),
            pltpu.VMEM((BW, 128), jnp.float32),
            pltpu.VMEM((BW, 128), jnp.float32),
            pltpu.VMEM((BW, 128), jnp.float32),
            pltpu.VMEM((D, BW), jnp.float32),
            pltpu.VMEM((D, BW), jnp.float32),
            pltpu.VMEM((D, BW), jnp.float32),
            pltpu.VMEM((D, BW), jnp.float32),
            pltpu.SemaphoreType.DMA,
            pltpu.SemaphoreType.DMA,
            pltpu.SemaphoreType.DMA,
            pltpu.SemaphoreType.DMA,
            pltpu.SemaphoreType.DMA,
            pltpu.SemaphoreType.DMA,
        ],
    )
    out_t = run(seq_t, ids_t, table2)
    return jnp.transpose(out_t, (2, 0, 1))


# speedup vs baseline: 1.2931x; 1.2931x over previous
import jax
import jax.numpy as jnp
from jax import lax
from jax.experimental import pallas as pl
from jax.experimental.pallas import tpu as pltpu
from jax.experimental.pallas import tpu_sc as plsc

D = 64
NC = 2
NS = 16
NW = NC * NS
BW = 128
LG = 8
BC = 8
CPOS = LG * BC
NCH = BW // BC


def _sc_body(seq_hbm, ids_hbm, table_hbm, out_hbm,
             slab_v,
             idx1_0, idx1_1, idx2_0, idx2_1, w1_0, w1_1, w2_0, w2_1,
             rows1_0, rows1_1, rows2_0, rows2_1,
             seq_0, seq_1, outb_0, outb_1,
             sem_g0, sem_g1, sem_s0, sem_s1, sem_o0, sem_o1):
    idx1_v = (idx1_0, idx1_1)
    idx2_v = (idx2_0, idx2_1)
    w1_v = (w1_0, w1_1)
    w2_v = (w2_0, w2_1)
    rows1_v = (rows1_0, rows1_1)
    rows2_v = (rows2_0, rows2_1)
    seq_v = (seq_0, seq_1)
    out_v = (outb_0, outb_1)
    sem_g = (sem_g0, sem_g1)
    sem_s = (sem_s0, sem_s1)
    sem_o = (sem_o0, sem_o1)

    wid = lax.axis_index("s") * NC + lax.axis_index("c")
    wb = wid * BW
    ngroups = ids_hbm.shape[0] // LG

    iota = lax.iota(jnp.int32, 16)
    zeros16 = iota * 0
    ones16 = zeros16 + 1
    li_lo = lax.shift_right_logical(iota, 3)
    bi16 = lax.bitwise_and(iota, zeros16 + 7)

    def stage_slab(g):
        pltpu.sync_copy(ids_hbm.at[pl.ds(g * LG, LG), :, :, pl.ds(wb, BW)],
                        slab_v)

    def prefetch(g, k, b):
        for v in range(CPOS // 16):
            li = li_lo + 2 * v
            bv = bi16 + k * BC
            sl = pl.ds(v * 16, 16)
            idx1_v[b][sl] = plsc.load_gather(slab_v, [li, zeros16, zeros16, bv])
            w1_v[b][sl] = plsc.load_gather(slab_v, [li, zeros16, ones16, bv])
            idx2_v[b][sl] = plsc.load_gather(slab_v, [li, ones16, zeros16, bv])
            w2_v[b][sl] = plsc.load_gather(slab_v, [li, ones16, ones16, bv])
        pltpu.async_copy(table_hbm.at[idx1_v[b]], rows1_v[b], sem_g[b])
        pltpu.async_copy(table_hbm.at[idx2_v[b]], rows2_v[b], sem_g[b])
        b0 = wb + k * BC
        l0 = g * LG
        pltpu.async_copy(seq_hbm.at[pl.ds(b0, BC), pl.ds(l0, LG)], seq_v[b],
                         sem_s[b])

    def wait_in(b):
        pltpu.make_async_copy(table_hbm.at[idx1_v[b]], rows1_v[b], sem_g[b]).wait()
        pltpu.make_async_copy(table_hbm.at[idx2_v[b]], rows2_v[b], sem_g[b]).wait()
        pltpu.make_async_copy(seq_hbm.at[pl.ds(0, BC), pl.ds(0, LG)], seq_v[b],
                              sem_s[b]).wait()

    def wait_out(b):
        pltpu.make_async_copy(out_v[b], out_hbm.at[pl.ds(0, BC), pl.ds(0, LG)],
                              sem_o[b]).wait()

    def compute(g, k, b):
        wait_in(b)

        @pl.when(g * NCH + k >= 2)
        def _():
            wait_out(b)

        r1, r2, sq, ob = rows1_v[b], rows2_v[b], seq_v[b], out_v[b]
        wv1, wv2 = w1_v[b], w2_v[b]

        @plsc.parallel_loop(0, CPOS, unroll=4)
        def pos(p):
            bi = lax.bitwise_and(p, BC - 1)
            li = lax.shift_right_logical(p, 3)
            ws1 = plsc.load_gather(wv1, [zeros16 + p]).astype(jnp.float32)
            ws2 = plsc.load_gather(wv2, [zeros16 + p]).astype(jnp.float32)
            for d in range(D // 16):
                sl = pl.ds(d * 16, 16)
                ob[bi, li, sl] = (sq[bi, li, sl] + r1[p, sl] * ws1
                                  + r2[p, sl] * ws2)

        b0 = wb + k * BC
        l0 = g * LG
        pltpu.async_copy(ob, out_hbm.at[pl.ds(b0, BC), pl.ds(l0, LG)], sem_o[b])

    def body(g, _):
        stage_slab(g)
        prefetch(g, 0, 0)
        prefetch(g, 1, 1)

        def inner(k2, _):
            k = k2 * 2
            compute(g, k, 0)
            prefetch(g, k + 2, 0)
            compute(g, k + 1, 1)
            prefetch(g, k + 3, 1)
            return 0

        lax.fori_loop(0, NCH // 2 - 1, inner, 0)
        compute(g, NCH - 2, 0)
        compute(g, NCH - 1, 1)
        return 0

    lax.fori_loop(0, ngroups, body, 0)

    wait_out(0)
    wait_out(1)


def kernel(sequence, id, player_embeddings):
    b, l, d = sequence.shape
    ids_t = jnp.transpose(id.astype(jnp.int32), (1, 2, 3, 0))
    table2 = jnp.pad(player_embeddings, ((0, 0), (0, 128 - d)))
    mesh = plsc.VectorSubcoreMesh(core_axis_name="c", subcore_axis_name="s")
    run = pl.kernel(
        _sc_body,
        out_type=jax.ShapeDtypeStruct((b, l, d), jnp.float32),
        mesh=mesh,
        compiler_params=pltpu.CompilerParams(needs_layout_passes=False),
        scratch_types=[
            pltpu.VMEM((LG, 2, 2, BW), jnp.int32),
            pltpu.VMEM((CPOS,), jnp.int32),
            pltpu.VMEM((CPOS,), jnp.int32),
            pltpu.VMEM((CPOS,), jnp.int32),
            pltpu.VMEM((CPOS,), jnp.int32),
            pltpu.VMEM((CPOS,), jnp.int32),
            pltpu.VMEM((CPOS,), jnp.int32),
            pltpu.VMEM((CPOS,), jnp.int32),
            pltpu.VMEM((CPOS,), jnp.int32),
            pltpu.VMEM((CPOS, 128), jnp.float32),
            pltpu.VMEM((CPOS, 128), jnp.float32),
            pltpu.VMEM((CPOS, 128), jnp.float32),
            pltpu.VMEM((CPOS, 128), jnp.float32),
            pltpu.VMEM((BC, LG, D), jnp.float32),
            pltpu.VMEM((BC, LG, D), jnp.float32),
            pltpu.VMEM((BC, LG, D), jnp.float32),
            pltpu.VMEM((BC, LG, D), jnp.float32),
            pltpu.SemaphoreType.DMA,
            pltpu.SemaphoreType.DMA,
            pltpu.SemaphoreType.DMA,
            pltpu.SemaphoreType.DMA,
            pltpu.SemaphoreType.DMA,
            pltpu.SemaphoreType.DMA,
        ],
    )
    return run(sequence, ids_t, table2)
